# baseline (speedup 1.0000x reference)
import jax
import jax.numpy as jnp
from jax.experimental import pallas as pl
from jax.experimental.pallas import tpu as pltpu

_NC = 1000
_SMOOTHING = 0.1
_CONF = 1.0 - _SMOOTHING
_EPS = _SMOOTHING / (_NC - 1)
_CHUNK = 1024
_N = 16384


def _outer(pred_hbm, tgt_hbm, out_ref, acc_ref):
    acc_ref[...] = jnp.zeros((1, 1), jnp.float32)

    def _chunk_body(pred_ref, tgt_ref):
        x = pred_ref[...]
        t = tgt_ref[...]
        m = jnp.max(x, axis=1, keepdims=True)
        s = jnp.sum(jnp.exp(x - m), axis=1, keepdims=True)
        lse = m + jnp.log(s)
        sum_pred = jnp.sum(x, axis=1, keepdims=True)
        col = jax.lax.broadcasted_iota(jnp.int32, (1, _NC), 1)
        p_t = jnp.sum(jnp.where(col == t, x, 0.0), axis=1, keepdims=True)
        blk = jnp.sum(lse - _EPS * sum_pred - (_CONF - _EPS) * p_t)
        acc_ref[...] += blk.reshape(1, 1)

    pltpu.emit_pipeline(
        _chunk_body,
        grid=(_N // _CHUNK,),
        in_specs=[
            pl.BlockSpec((_CHUNK, _NC), lambda i: (i, 0)),
            pl.BlockSpec((_CHUNK, 1), lambda i: (i, 0)),
        ],
    )(pred_hbm, tgt_hbm)
    out_ref[...] = acc_ref[...] * (1.0 / _N)


def kernel(pred, target):
    n = target.shape[0]
    tgt2d = target.astype(jnp.int32).reshape(n, 1)
    total = pl.pallas_call(
        _outer,
        in_specs=[
            pl.BlockSpec(memory_space=pl.ANY),
            pl.BlockSpec(memory_space=pl.ANY),
        ],
        out_specs=pl.BlockSpec(memory_space=pltpu.MemorySpace.VMEM),
        out_shape=jax.ShapeDtypeStruct((1, 1), jnp.float32),
        scratch_shapes=[pltpu.VMEM((1, 1), jnp.float32)],
    )(pred, tgt2d)
    return total[0, 0]


# speedup vs baseline: 1.0501x; 1.0501x over previous
import jax
import jax.numpy as jnp
from jax.experimental import pallas as pl
from jax.experimental.pallas import tpu as pltpu

_NC = 1000
_SMOOTHING = 0.1
_CONF = 1.0 - _SMOOTHING
_EPS = _SMOOTHING / (_NC - 1)
_NWIN = 4
_BLK = 512
_N = 16384
_STEPS = _N // (_NWIN * _BLK)


def _loss_block(*refs):
    out_ref = refs[-1]
    pred_refs = refs[:_NWIN]
    tgt_refs = refs[_NWIN:2 * _NWIN]
    i = pl.program_id(0)
    ng = pl.num_programs(0)

    col = jax.lax.broadcasted_iota(jnp.int32, (1, _NC), 1)
    blk = jnp.zeros((), jnp.float32)
    for pref, tref in zip(pred_refs, tgt_refs):
        x = pref[...]
        t = tref[...]
        m = jnp.max(x, axis=1, keepdims=True)
        s = jnp.sum(jnp.exp(x - m), axis=1, keepdims=True)
        lse = m + jnp.log(s)
        sum_pred = jnp.sum(x, axis=1, keepdims=True)
        p_t = jnp.sum(jnp.where(col == t, x, 0.0), axis=1, keepdims=True)
        blk += jnp.sum(lse - _EPS * sum_pred - (_CONF - _EPS) * p_t)

    @pl.when(i == 0)
    def _init():
        out_ref[...] = jnp.zeros((1, 1), jnp.float32)

    out_ref[...] += blk.reshape(1, 1)

    @pl.when(i == ng - 1)
    def _final():
        out_ref[...] = out_ref[...] * (1.0 / _N)


def kernel(pred, target):
    n = target.shape[0]
    tgt2d = target.astype(jnp.int32).reshape(n, 1)
    pred_specs = [
        pl.BlockSpec((_BLK, _NC), (lambda i, k=k: (i + k * _STEPS, 0)))
        for k in range(_NWIN)
    ]
    tgt_specs = [
        pl.BlockSpec((_BLK, 1), (lambda i, k=k: (i + k * _STEPS, 0)))
        for k in range(_NWIN)
    ]
    total = pl.pallas_call(
        _loss_block,
        grid=(_STEPS,),
        in_specs=pred_specs + tgt_specs,
        out_specs=pl.BlockSpec((1, 1), lambda i: (0, 0)),
        out_shape=jax.ShapeDtypeStruct((1, 1), jnp.float32),
    )(*([pred] * _NWIN + [tgt2d] * _NWIN))
    return total[0, 0]
